# baseline (speedup 1.0000x reference)
import functools

import jax
import jax.numpy as jnp
from jax import lax
from jax.experimental import pallas as pl
from jax.experimental.pallas import tpu as pltpu
from jax.experimental.pallas import tpu_sc as plsc

_B = 128
_E = 128
_W = 5
_NNEG = 5
_NC = 2
_NSC = 16
_VPW = _B // _NSC
_OPW = _B // (_NC * _NSC)
_IPW = 72
_NL = 16
_NCH = _E // _NL
_XOFF, _LOFF, _NOFF = 0, 40, 48

_mesh = plsc.VectorSubcoreMesh(core_axis_name="c", subcore_axis_name="s")


def _allsum16(v):
    iota = lax.iota(jnp.int32, _NL)
    for sh in (1, 2, 4, 8):
        v = v + v.at[jnp.bitwise_xor(iota, sh)].get(mode="promise_in_bounds")
    return v


def _softplus16(t):
    u = jnp.exp(-jnp.abs(t))
    z = u / (u + 2.0)
    z2 = z * z
    ln1p = 2.0 * z * (1.0 + z2 * (1.0 / 3.0 + z2 * (0.2 + z2 * (1.0 / 7.0))))
    return jnp.maximum(t, 0.0) + ln1p


@functools.partial(
    pl.kernel,
    mesh=_mesh,
    out_type=jax.ShapeDtypeStruct((_NC, _NL), jnp.float32),
    compiler_params=pltpu.CompilerParams(needs_layout_passes=False),
    scratch_types=[
        pltpu.VMEM((_IPW,), jnp.int32),
        pltpu.VMEM((_IPW, _E), jnp.float32),
        pltpu.VMEM((_VPW * _E,), jnp.float32),
        pltpu.VMEM((_B * _E,), jnp.float32),
        pltpu.VMEM((_NL,), jnp.float32),
        pltpu.SMEM((1,), jnp.int32),
        pltpu.VMEM_SHARED((_B * _E,), jnp.float32),
        pltpu.SemaphoreType.DMA,
        pltpu.SemaphoreType.DMA,
    ],
)
def _sc_loss(idx_hbm, table_hbm, out_hbm,
             idx_v, rows_v, vp_v, vp_all, loss_v, acc_smem, vp_sh, sem,
             sem2):
    core = lax.axis_index("c")
    sid = lax.axis_index("s")
    wid = core * _NSC + sid
    acc_smem[0] = 0
    pltpu.sync_copy(idx_hbm.at[pl.ds(wid * _IPW, _IPW)], idx_v)
    cp_x = pltpu.async_copy(table_hbm.at[idx_v.at[pl.ds(_XOFF, 40)]],
                            rows_v.at[pl.ds(_XOFF, 40)], sem)
    cp_l = pltpu.async_copy(table_hbm.at[idx_v.at[pl.ds(_LOFF, 8)]],
                            rows_v.at[pl.ds(_LOFF, 8)], sem2)
    cp_n = pltpu.async_copy(table_hbm.at[idx_v.at[pl.ds(_NOFF, 24)]],
                            rows_v.at[pl.ds(_NOFF, 24)], sem2)
    cp_x.wait()

    inv = jnp.float32(1.0 / _W)
    iota = lax.iota(jnp.int32, _NL)
    fzero = jnp.zeros((_NL,), jnp.float32)

    def _pa_c(c, j):
        sl = pl.ds(c * _NL, _NL)
        vp = rows_v[_XOFF + _W * j, sl]
        for t in range(1, _W):
            vp = vp + rows_v[_XOFF + _W * j + t, sl]
        vp_v[pl.ds(j * _E + c * _NL, _NL)] = vp * inv
        return j

    def _pa_j(j, carry):
        lax.fori_loop(0, _NCH, _pa_c, j)
        return carry

    lax.fori_loop(0, _VPW, _pa_j, 0)
    pltpu.sync_copy(vp_v, vp_sh.at[pl.ds(sid * _VPW * _E, _VPW * _E)])
    cp_l.wait()
    cp_n.wait()
    plsc.subcore_barrier()
    cp_vp = pltpu.async_copy(vp_sh, vp_all, sem)

    def _pb_c(c, acc):
        j, acc2 = acc
        sl = pl.ds(c * _NL, _NL)
        vn = rows_v[_NOFF + _NNEG * j, sl]
        for t in range(1, _NNEG):
            vn = vn + rows_v[_NOFF + _NNEG * j + t, sl]
        return (j, acc2 + rows_v[_LOFF + j, sl] * (vn * inv))

    def _pb_j(j, v):
        _, acc2 = lax.fori_loop(0, _NCH, _pb_c, (j, fzero))
        return jnp.where(iota == j + _OPW, _allsum16(acc2), v)

    v2 = lax.fori_loop(0, _OPW, _pb_j, fzero)
    cp_vp.wait()

    def _s1_c(c, acc):
        j, col, acc1 = acc
        flat = (iota + c * _NL) * _E + col
        g = plsc.load_gather(vp_all, [flat])
        return (j, col, acc1 + rows_v[_LOFF + j, pl.ds(c * _NL, _NL)] * g)

    def _s1_j(j, v):
        col = jnp.full((_NL,), wid * _OPW + j, jnp.int32)
        _, _, acc1 = lax.fori_loop(0, _NCH, _s1_c, (j, col, fzero))
        return jnp.where(iota == j, -_allsum16(acc1), v)

    v1 = lax.fori_loop(0, _OPW, _s1_j, fzero)

    sp = jnp.where(iota < 2 * _OPW, _softplus16(v1 + v2), fzero)

    psum = _allsum16(sp)
    pi = ((psum * jnp.float32(1048576.0))
          + jnp.float32(0.5)).astype(jnp.int32)
    plsc.subcore_barrier()
    plsc.fetch_and_add(acc_smem.at[0], pi[0], subcore_id=0)
    plsc.subcore_barrier()

    @pl.when(sid == 0)
    def _():
        tot = acc_smem[0]
        part = tot.astype(jnp.float32) * jnp.float32(1.0 / (1048576.0 * _B))
        loss_v[...] = jnp.full((_NL,), part, jnp.float32)
        pltpu.sync_copy(loss_v, out_hbm.at[core])


def kernel(x, label, negs, table):
    xr = x.reshape(_NSC, _VPW * _W)
    xr32 = jnp.concatenate([xr, xr], axis=0)
    lab32 = label.reshape(_NC * _NSC, _OPW)
    nr32 = negs.reshape(_NC * _NSC, _OPW * _NNEG)
    pad = jnp.zeros((_NC * _NSC, 4), jnp.int32)
    idx = jnp.concatenate([xr32, lab32, pad, nr32, pad], axis=1).reshape(-1)
    out = _sc_loss(idx, table)
    return out[0, 0] + out[1, 0]


# speedup vs baseline: 1.4856x; 1.4856x over previous
import functools

import jax
import jax.numpy as jnp
from jax import lax
from jax.experimental import pallas as pl
from jax.experimental.pallas import tpu as pltpu
from jax.experimental.pallas import tpu_sc as plsc

_B = 128
_E = 128
_W = 5
_NNEG = 5
_NWK = 16
_BPW = _B // _NWK
_RPW = _BPW * (1 + _W + _NNEG)
_NL = 16
_NCH = _E // _NL

_mesh = plsc.VectorSubcoreMesh(
    core_axis_name="c", subcore_axis_name="s", num_cores=1)


def _allsum16(v):
    iota = lax.iota(jnp.int32, _NL)
    for sh in (1, 2, 4, 8):
        v = v + v.at[jnp.bitwise_xor(iota, sh)].get(mode="promise_in_bounds")
    return v


def _softplus16(t):
    u = jnp.exp(-jnp.abs(t))
    z = u / (u + 2.0)
    z2 = z * z
    ln1p = 2.0 * z * (1.0 + z2 * (1.0 / 3.0 + z2 * (0.2 + z2 * (1.0 / 7.0))))
    return jnp.maximum(t, 0.0) + ln1p


@functools.partial(
    pl.kernel,
    mesh=_mesh,
    out_type=jax.ShapeDtypeStruct((_NL,), jnp.float32),
    compiler_params=pltpu.CompilerParams(needs_layout_passes=False),
    scratch_types=[
        pltpu.VMEM((_RPW + 16,), jnp.int32),
        pltpu.VMEM((_BPW, _W), jnp.int32),
        pltpu.VMEM((_BPW, _NNEG), jnp.int32),
        pltpu.VMEM((_RPW + 16, _E), jnp.float32),
        pltpu.VMEM((_BPW * _E,), jnp.float32),
        pltpu.VMEM((_B * _E,), jnp.float32),
        pltpu.VMEM((_NL,), jnp.float32),
        pltpu.SMEM((1,), jnp.int32),
        pltpu.VMEM_SHARED((_B * _E,), jnp.float32),
        pltpu.SemaphoreType.DMA,
        pltpu.SemaphoreType.DMA,
    ],
)
def _sc_loss(x_hbm, label_hbm, negs_hbm, table_hbm, out_hbm,
             idx_v, x_i2, n_i2, rows_v, vp_v, vp_all, loss_v, acc_smem,
             vp_sh, sem, sem2):
    wid = lax.axis_index("s")
    acc_smem[0] = 0
    iota = lax.iota(jnp.int32, _NL)
    bsl = pl.ds(wid * _BPW, _BPW)
    pltpu.sync_copy(label_hbm.at[bsl], idx_v.at[pl.ds(0, _BPW)])
    pltpu.sync_copy(x_hbm.at[bsl, :], x_i2)
    pltpu.sync_copy(negs_hbm.at[bsl, :], n_i2)
    for t in range(3):
        p = jnp.minimum(iota + t * _NL, jnp.int32(_BPW * _W - 1))
        gx = plsc.load_gather(x_i2, [p // _W, p % _W])
        idx_v[pl.ds(_BPW + t * _NL, _NL)] = gx
        gn = plsc.load_gather(n_i2, [p // _NNEG, p % _NNEG])
        idx_v[pl.ds(_BPW + 48 + t * _NL, _NL)] = gn
    cp_x = pltpu.async_copy(table_hbm.at[idx_v.at[pl.ds(_BPW, 48)]],
                            rows_v.at[pl.ds(_BPW, 48)], sem)
    cp_l = pltpu.async_copy(table_hbm.at[idx_v.at[pl.ds(0, _BPW)]],
                            rows_v.at[pl.ds(0, _BPW)], sem2)
    cp_n = pltpu.async_copy(
        table_hbm.at[idx_v.at[pl.ds(_BPW + 48, 48)]],
        rows_v.at[pl.ds(_BPW + 48, 48)], sem2)
    cp_x.wait()

    inv = jnp.float32(1.0 / _W)
    fzero = jnp.zeros((_NL,), jnp.float32)

    def _pa_c(c, j):
        sl = pl.ds(c * _NL, _NL)
        vp = rows_v[_BPW + _W * j, sl]
        for t in range(1, _W):
            vp = vp + rows_v[_BPW + _W * j + t, sl]
        vp_v[pl.ds(j * _E + c * _NL, _NL)] = vp * inv
        return j

    def _pa_j(j, carry):
        lax.fori_loop(0, _NCH, _pa_c, j)
        return carry

    lax.fori_loop(0, _BPW, _pa_j, 0)
    pltpu.sync_copy(vp_v, vp_sh.at[pl.ds(wid * _BPW * _E, _BPW * _E)])
    cp_l.wait()
    cp_n.wait()
    plsc.subcore_barrier()
    cp_vp = pltpu.async_copy(vp_sh, vp_all, sem)

    base_n = _BPW + 48

    def _pb_c(c, acc):
        j, acc2 = acc
        sl = pl.ds(c * _NL, _NL)
        vn = rows_v[base_n + _NNEG * j, sl]
        for t in range(1, _NNEG):
            vn = vn + rows_v[base_n + _NNEG * j + t, sl]
        return (j, acc2 + rows_v[j, sl] * (vn * inv))

    def _pb_j(j, v):
        _, acc2 = lax.fori_loop(0, _NCH, _pb_c, (j, fzero))
        return jnp.where(iota == j + _BPW, _allsum16(acc2), v)

    v2 = lax.fori_loop(0, _BPW, _pb_j, fzero)
    cp_vp.wait()

    def _s1_c(c, acc):
        j, col, acc1 = acc
        flat = (iota + c * _NL) * _E + col
        g = plsc.load_gather(vp_all, [flat])
        return (j, col, acc1 + rows_v[j, pl.ds(c * _NL, _NL)] * g)

    def _s1_j(j, v):
        col = jnp.full((_NL,), wid * _BPW + j, jnp.int32)
        _, _, acc1 = lax.fori_loop(0, _NCH, _s1_c, (j, col, fzero))
        return jnp.where(iota == j, -_allsum16(acc1), v)

    v1 = lax.fori_loop(0, _BPW, _s1_j, fzero)

    sp = _softplus16(v1 + v2)

    psum = _allsum16(sp)
    pi = ((psum * jnp.float32(1048576.0))
          + jnp.float32(0.5)).astype(jnp.int32)
    plsc.subcore_barrier()
    plsc.fetch_and_add(acc_smem.at[0], pi[0], subcore_id=0)
    plsc.subcore_barrier()

    @pl.when(wid == 0)
    def _():
        tot = acc_smem[0]
        loss = tot.astype(jnp.float32) * jnp.float32(1.0 / (1048576.0 * _B))
        loss_v[...] = jnp.full((_NL,), loss, jnp.float32)
        pltpu.sync_copy(loss_v, out_hbm)


def kernel(x, label, negs, table):
    out = _sc_loss(x, label, negs, table)
    return out[0]


# speedup vs baseline: 1.5576x; 1.0485x over previous
import functools

import jax
import jax.numpy as jnp
from jax import lax
from jax.experimental import pallas as pl
from jax.experimental.pallas import tpu as pltpu
from jax.experimental.pallas import tpu_sc as plsc

_B = 128
_E = 128
_W = 5
_NNEG = 5
_NWK = 16
_BPW = _B // _NWK
_RPW = _BPW * (1 + _W + _NNEG)
_NL = 16
_NCH = _E // _NL

_mesh = plsc.VectorSubcoreMesh(
    core_axis_name="c", subcore_axis_name="s", num_cores=1)


@functools.partial(
    pl.kernel,
    mesh=_mesh,
    out_type=(
        jax.ShapeDtypeStruct((_B, _E), jnp.float32),
        jax.ShapeDtypeStruct((_B, _E), jnp.float32),
        jax.ShapeDtypeStruct((_B, _E), jnp.float32),
    ),
    compiler_params=pltpu.CompilerParams(needs_layout_passes=False),
    scratch_types=[
        pltpu.VMEM((_RPW,), jnp.int32),
        pltpu.VMEM((_RPW, _E), jnp.float32),
        pltpu.VMEM((_BPW, _E), jnp.float32),
        pltpu.VMEM((_BPW, _E), jnp.float32),
        pltpu.SemaphoreType.DMA,
        pltpu.SemaphoreType.DMA,
    ],
)
def _sc_gather_mean(idx_hbm, table_hbm, u_out, vp_out, vn_out,
                    idx_v, rows_v, vp_v, vn_v, sem, sem2):
    wid = lax.axis_index("s")
    pltpu.sync_copy(idx_hbm.at[pl.ds(wid * _RPW, _RPW)], idx_v)
    _nx = _BPW * _W
    cp_x = pltpu.async_copy(table_hbm.at[idx_v.at[pl.ds(_BPW, _nx)]],
                            rows_v.at[pl.ds(_BPW, _nx)], sem)
    cp_l = pltpu.async_copy(table_hbm.at[idx_v.at[pl.ds(0, _BPW)]],
                            rows_v.at[pl.ds(0, _BPW)], sem2)
    cp_n = pltpu.async_copy(
        table_hbm.at[idx_v.at[pl.ds(_BPW + _nx, _BPW * _NNEG)]],
        rows_v.at[pl.ds(_BPW + _nx, _BPW * _NNEG)], sem2)
    cp_x.wait()

    inv = jnp.float32(1.0 / _W)
    rsl = pl.ds(wid * _BPW, _BPW)

    def _pa_c(c, j):
        sl = pl.ds(c * _NL, _NL)
        vp = rows_v[_BPW + _W * j, sl]
        for t in range(1, _W):
            vp = vp + rows_v[_BPW + _W * j + t, sl]
        vp_v[j, sl] = vp * inv
        return j

    def _pa_j(j, carry):
        lax.fori_loop(0, _NCH, _pa_c, j)
        return carry

    lax.fori_loop(0, _BPW, _pa_j, 0)
    cp_vp = pltpu.async_copy(vp_v, vp_out.at[rsl], sem)
    cp_l.wait()
    cp_n.wait()
    cp_u = pltpu.async_copy(rows_v.at[pl.ds(0, _BPW)], u_out.at[rsl], sem2)

    base_n = _BPW * (1 + _W)

    def _pb_c(c, j):
        sl = pl.ds(c * _NL, _NL)
        vn = rows_v[base_n + _NNEG * j, sl]
        for t in range(1, _NNEG):
            vn = vn + rows_v[base_n + _NNEG * j + t, sl]
        vn_v[j, sl] = vn * inv
        return j

    def _pb_j(j, carry):
        lax.fori_loop(0, _NCH, _pb_c, j)
        return carry

    lax.fori_loop(0, _BPW, _pb_j, 0)
    pltpu.sync_copy(vn_v, vn_out.at[rsl])
    cp_vp.wait()
    cp_u.wait()


def _tc_loss_body(u_ref, vp_ref, vn_ref, out_ref):
    u = u_ref[...]
    vp = vp_ref[...]
    vn = vn_ref[...]
    m = jnp.dot(u, vp, preferred_element_type=jnp.float32)
    ri = lax.broadcasted_iota(jnp.int32, (_B, _B), 0)
    ci = lax.broadcasted_iota(jnp.int32, (_B, _B), 1)
    s1 = jnp.sum(jnp.where(ri == ci, m, 0.0), axis=1, keepdims=True)
    s2 = jnp.sum(u * vn, axis=1, keepdims=True)
    l1 = -jnp.log(1.0 / (1.0 + jnp.exp(-s1)))
    l2 = -jnp.log(1.0 / (1.0 + jnp.exp(s2)))
    out_ref[...] = jnp.reshape(jnp.mean(l1) + jnp.mean(l2), (1, 1))


_tc_loss = pl.pallas_call(
    _tc_loss_body,
    out_shape=jax.ShapeDtypeStruct((1, 1), jnp.float32),
)


def kernel(x, label, negs, table):
    lab = label.reshape(_NWK, _BPW)
    xr = x.reshape(_NWK, _BPW * _W)
    nr = negs.reshape(_NWK, _BPW * _NNEG)
    idx = jnp.concatenate([lab, xr, nr], axis=1).reshape(-1)
    u, vp, vn = _sc_gather_mean(idx, table)
    loss = _tc_loss(u, vp, vn)
    return loss[0, 0]


# speedup vs baseline: 1.5768x; 1.0123x over previous
import functools

import jax
import jax.numpy as jnp
from jax import lax
from jax.experimental import pallas as pl
from jax.experimental.pallas import tpu as pltpu
from jax.experimental.pallas import tpu_sc as plsc

_B = 128
_E = 128
_W = 5
_NNEG = 5
_NWK = 16
_BPW = _B // _NWK
_RPW = _BPW * (1 + _W + _NNEG)
_NL = 16
_NCH = _E // _NL

_mesh = plsc.VectorSubcoreMesh(
    core_axis_name="c", subcore_axis_name="s", num_cores=1)


def _allsum16(v):
    iota = lax.iota(jnp.int32, _NL)
    for sh in (1, 2, 4, 8):
        v = v + v.at[jnp.bitwise_xor(iota, sh)].get(mode="promise_in_bounds")
    return v


def _softplus16(t):
    u = jnp.exp(-jnp.abs(t))
    z = u / (u + 2.0)
    z2 = z * z
    ln1p = 2.0 * z * (1.0 + z2 * (1.0 / 3.0 + z2 * (0.2 + z2 * (1.0 / 7.0))))
    return jnp.maximum(t, 0.0) + ln1p


@functools.partial(
    pl.kernel,
    mesh=_mesh,
    out_type=jax.ShapeDtypeStruct((_NL,), jnp.float32),
    compiler_params=pltpu.CompilerParams(needs_layout_passes=False),
    scratch_types=[
        pltpu.VMEM((_RPW,), jnp.int32),
        pltpu.VMEM((_RPW, _E), jnp.float32),
        pltpu.VMEM((_BPW, _E), jnp.float32),
        pltpu.VMEM((_B, _E), jnp.float32),
        pltpu.VMEM((_NL,), jnp.float32),
        pltpu.SMEM((1,), jnp.int32),
        pltpu.VMEM_SHARED((_B, _E), jnp.float32),
        pltpu.SemaphoreType.DMA,
        pltpu.SemaphoreType.DMA,
    ],
)
def _sc_loss(idx_hbm, table_hbm, out_hbm,
             idx_v, rows_v, vp_v, vp_cols, loss_v, acc_smem, vp_sh, sem,
             sem2):
    wid = lax.axis_index("s")
    acc_smem[0] = 0
    pltpu.sync_copy(idx_hbm.at[pl.ds(wid * _RPW, _RPW)], idx_v)
    _nx = _BPW * _W
    cp_x = pltpu.async_copy(table_hbm.at[idx_v.at[pl.ds(_BPW, _nx)]],
                            rows_v.at[pl.ds(_BPW, _nx)], sem)
    cp_l = pltpu.async_copy(table_hbm.at[idx_v.at[pl.ds(0, _BPW)]],
                            rows_v.at[pl.ds(0, _BPW)], sem2)
    cp_n = pltpu.async_copy(
        table_hbm.at[idx_v.at[pl.ds(_BPW + _nx, _BPW * _NNEG)]],
        rows_v.at[pl.ds(_BPW + _nx, _BPW * _NNEG)], sem2)
    cp_x.wait()

    inv = jnp.float32(1.0 / _W)
    iota = lax.iota(jnp.int32, _NL)
    fzero = jnp.zeros((_NL,), jnp.float32)

    def _pa_c(c, j):
        sl = pl.ds(c * _NL, _NL)
        vp = rows_v[_BPW + _W * j, sl]
        for t in range(1, _W):
            vp = vp + rows_v[_BPW + _W * j + t, sl]
        vp_v[j, sl] = vp * inv
        return j

    def _pa_j(j, carry):
        lax.fori_loop(0, _NCH, _pa_c, j)
        return carry

    lax.fori_loop(0, _BPW, _pa_j, 0)
    pltpu.sync_copy(vp_v, vp_sh.at[pl.ds(wid * _BPW, _BPW), :])
    cp_l.wait()
    cp_n.wait()
    plsc.subcore_barrier()
    csl = pl.ds(wid * _BPW, _BPW)
    cp_vp = pltpu.async_copy(vp_sh.at[:, csl], vp_cols.at[:, csl], sem)

    base_n = _BPW * (1 + _W)

    def _pb_c(c, acc):
        j, acc2 = acc
        sl = pl.ds(c * _NL, _NL)
        vn = rows_v[base_n + _NNEG * j, sl]
        for t in range(1, _NNEG):
            vn = vn + rows_v[base_n + _NNEG * j + t, sl]
        return (j, acc2 + rows_v[j, sl] * (vn * inv))

    def _pb_j(j, v):
        _, acc2 = lax.fori_loop(0, _NCH, _pb_c, (j, fzero))
        return jnp.where(iota == j + _BPW, _allsum16(acc2), v)

    v2 = lax.fori_loop(0, _BPW, _pb_j, fzero)
    cp_vp.wait()

    def _s1_c(c, acc):
        j, col, acc1 = acc
        g = plsc.load_gather(vp_cols, [iota + c * _NL, col])
        return (j, col, acc1 + rows_v[j, pl.ds(c * _NL, _NL)] * g)

    def _s1_j(j, v):
        col = jnp.full((_NL,), wid * _BPW + j, jnp.int32)
        _, _, acc1 = lax.fori_loop(0, _NCH, _s1_c, (j, col, fzero))
        return jnp.where(iota == j, -_allsum16(acc1), v)

    v1 = lax.fori_loop(0, _BPW, _s1_j, fzero)

    sp = _softplus16(v1 + v2)

    psum = _allsum16(sp)
    pi = ((psum * jnp.float32(1048576.0))
          + jnp.float32(0.5)).astype(jnp.int32)
    plsc.subcore_barrier()
    plsc.fetch_and_add(acc_smem.at[0], pi[0], subcore_id=0)
    plsc.subcore_barrier()

    @pl.when(wid == 0)
    def _():
        tot = acc_smem[0]
        loss = tot.astype(jnp.float32) * jnp.float32(1.0 / (1048576.0 * _B))
        loss_v[...] = jnp.full((_NL,), loss, jnp.float32)
        pltpu.sync_copy(loss_v, out_hbm)


def kernel(x, label, negs, table):
    lab = label.reshape(_NWK, _BPW)
    xr = x.reshape(_NWK, _BPW * _W)
    nr = negs.reshape(_NWK, _BPW * _NNEG)
    idx = jnp.concatenate([lab, xr, nr], axis=1).reshape(-1)
    out = _sc_loss(idx, table)
    return out[0]


# speedup vs baseline: 1.5924x; 1.0099x over previous
import functools

import jax
import jax.numpy as jnp
from jax import lax
from jax.experimental import pallas as pl
from jax.experimental.pallas import tpu as pltpu
from jax.experimental.pallas import tpu_sc as plsc

_B = 128
_E = 128
_W = 5
_NNEG = 5
_NWK = 16
_BPW = _B // _NWK
_RPW = _BPW * (1 + _W + _NNEG)
_NL = 16
_NCH = _E // _NL

_mesh = plsc.VectorSubcoreMesh(
    core_axis_name="c", subcore_axis_name="s", num_cores=1)


def _allsum16(v):
    iota = lax.iota(jnp.int32, _NL)
    for sh in (1, 2, 4, 8):
        v = v + v.at[jnp.bitwise_xor(iota, sh)].get(mode="promise_in_bounds")
    return v


def _softplus16(t):
    u = jnp.exp(-jnp.abs(t))
    z = u / (u + 2.0)
    z2 = z * z
    ln1p = 2.0 * z * (1.0 + z2 * (1.0 / 3.0 + z2 * (0.2 + z2 * (1.0 / 7.0))))
    return jnp.maximum(t, 0.0) + ln1p


@functools.partial(
    pl.kernel,
    mesh=_mesh,
    out_type=jax.ShapeDtypeStruct((_NL,), jnp.float32),
    compiler_params=pltpu.CompilerParams(needs_layout_passes=False),
    scratch_types=[
        pltpu.VMEM((_RPW,), jnp.int32),
        pltpu.VMEM((_RPW, _E), jnp.float32),
        pltpu.VMEM((_BPW * _E,), jnp.float32),
        pltpu.VMEM((_B * _E,), jnp.float32),
        pltpu.VMEM((_NL,), jnp.float32),
        pltpu.SMEM((1,), jnp.int32),
        pltpu.VMEM_SHARED((_B * _E,), jnp.float32),
        pltpu.SemaphoreType.DMA,
        pltpu.SemaphoreType.DMA,
    ],
)
def _sc_loss(idx_hbm, table_hbm, out_hbm,
             idx_v, rows_v, vp_v, vp_all, loss_v, acc_smem, vp_sh, sem,
             sem2):
    wid = lax.axis_index("s")
    acc_smem[0] = 0
    pltpu.sync_copy(idx_hbm.at[pl.ds(wid * _RPW, _RPW)], idx_v)
    _nx = _BPW * _W
    cp_x = pltpu.async_copy(table_hbm.at[idx_v.at[pl.ds(_BPW, _nx)]],
                            rows_v.at[pl.ds(_BPW, _nx)], sem)
    cp_l = pltpu.async_copy(table_hbm.at[idx_v.at[pl.ds(0, _BPW)]],
                            rows_v.at[pl.ds(0, _BPW)], sem2)
    cp_n = pltpu.async_copy(
        table_hbm.at[idx_v.at[pl.ds(_BPW + _nx, _BPW * _NNEG)]],
        rows_v.at[pl.ds(_BPW + _nx, _BPW * _NNEG)], sem2)
    cp_x.wait()

    inv = jnp.float32(1.0 / _W)
    iota = lax.iota(jnp.int32, _NL)
    fzero = jnp.zeros((_NL,), jnp.float32)

    def _pa_c(c, j):
        sl = pl.ds(c * _NL, _NL)
        vp = rows_v[_BPW + _W * j, sl]
        for t in range(1, _W):
            vp = vp + rows_v[_BPW + _W * j + t, sl]
        vp_v[pl.ds(j * _E + c * _NL, _NL)] = vp * inv
        return j

    def _pa_j(j, carry):
        lax.fori_loop(0, _NCH, _pa_c, j)
        return carry

    lax.fori_loop(0, _BPW, _pa_j, 0)
    pltpu.sync_copy(vp_v, vp_sh.at[pl.ds(wid * _BPW * _E, _BPW * _E)])
    cp_l.wait()
    cp_n.wait()
    plsc.subcore_barrier()
    cp_vp = pltpu.async_copy(vp_sh, vp_all, sem)

    base_n = _BPW * (1 + _W)

    def _pb_c(c, acc):
        j, acc2 = acc
        sl = pl.ds(c * _NL, _NL)
        vn = rows_v[base_n + _NNEG * j, sl]
        for t in range(1, _NNEG):
            vn = vn + rows_v[base_n + _NNEG * j + t, sl]
        return (j, acc2 + rows_v[j, sl] * (vn * inv))

    def _pb_j(j, v):
        _, acc2 = lax.fori_loop(0, _NCH, _pb_c, (j, fzero))
        return jnp.where(iota == j + _BPW, _allsum16(acc2), v)

    v2 = lax.fori_loop(0, _BPW, _pb_j, fzero)
    cp_vp.wait()

    def _s1_c(c, acc):
        j, col, acc1 = acc
        flat = (iota + c * _NL) * _E + col
        g = plsc.load_gather(vp_all, [flat])
        return (j, col, acc1 + rows_v[j, pl.ds(c * _NL, _NL)] * g)

    def _s1_j(j, v):
        col = jnp.full((_NL,), wid * _BPW + j, jnp.int32)
        _, _, acc1 = lax.fori_loop(0, _NCH, _s1_c, (j, col, fzero))
        return jnp.where(iota == j, -_allsum16(acc1), v)

    v1 = lax.fori_loop(0, _BPW, _s1_j, fzero)

    sp = _softplus16(v1 + v2)

    psum = _allsum16(sp)
    pi = ((psum * jnp.float32(1048576.0))
          + jnp.float32(0.5)).astype(jnp.int32)
    plsc.subcore_barrier()
    plsc.fetch_and_add(acc_smem.at[0], pi[0], subcore_id=0)
    plsc.subcore_barrier()

    @pl.when(wid == 0)
    def _():
        tot = acc_smem[0]
        loss = tot.astype(jnp.float32) * jnp.float32(1.0 / (1048576.0 * _B))
        loss_v[...] = jnp.full((_NL,), loss, jnp.float32)
        pltpu.sync_copy(loss_v, out_hbm)


def kernel(x, label, negs, table):
    lab = label.reshape(_NWK, _BPW)
    xr = x.reshape(_NWK, _BPW * _W)
    nr = negs.reshape(_NWK, _BPW * _NNEG)
    idx = jnp.concatenate([lab, xr, nr], axis=1).reshape(-1)
    out = _sc_loss(idx, table)
    return out[0]
